# baseline (speedup 1.0000x reference)
import functools

import jax
import jax.numpy as jnp
from jax import lax
from jax.experimental import pallas as pl
from jax.experimental.pallas import tpu as pltpu

N_CODES = 8192
DIM = 32
N_VEC = 8192
R = 256
C = 256
N_BLOCKS = N_VEC // R
N_CHUNKS = N_CODES // C
HALF = N_CODES // 2
CHUNKS_PER_HALF = HALF // C
BETA_ = 0.25
MM_PREC = lax.Precision.HIGHEST


def _tree_sum_lanes(v):
    k = v.shape[-1]
    while k > 1:
        k //= 2
        v = v[..., :k] + v[..., k:2 * k]
    return v


def _tree_sum_rows(v):
    k = v.shape[0]
    while k > 1:
        k //= 2
        v = v[:k, :] + v[k:2 * k, :]
    return v


def _vq_kernel(z_ref, emb_ref, embT_ref, embT2b_ref,
               enc_ref, zq_ref, idx_ref, loss_ref, perp_ref,
               counts_ref, acc_ref):
    i = pl.program_id(0)
    zb = z_ref[...]
    zb16 = zb.astype(jnp.bfloat16)
    zsq = _tree_sum_lanes(zb * zb)
    embT = embT_ref[...]
    esq = _tree_sum_rows(embT * embT)

    @pl.when(i == 0)
    def _init():
        counts_ref[...] = jnp.zeros((1, N_CODES), jnp.float32)
        acc_ref[...] = jnp.zeros((1, 1), jnp.float32)

    half_v = []
    half_i = []
    for h in range(2):
        m = jnp.full((R, 1), jnp.inf, jnp.float32)
        mi = jnp.zeros((R, 1), jnp.int32)
        for cc in range(CHUNKS_PER_HALF):
            c = h * CHUNKS_PER_HALF + cc
            et2 = embT2b_ref[:, c * C:(c + 1) * C]
            mm2 = jnp.dot(zb16, et2,
                          preferred_element_type=jnp.float32)
            d = (zsq + esq[:, c * C:(c + 1) * C]) - mm2
            mc = jnp.min(d, axis=1, keepdims=True)
            iota = lax.broadcasted_iota(jnp.int32, (R, C), 1) + c * C
            ic = jnp.min(jnp.where(d == mc, iota, jnp.int32(2**30)),
                         axis=1, keepdims=True)
            upd = mc < m
            m = jnp.where(upd, mc, m)
            mi = jnp.where(upd, ic, mi)
        half_v.append(m)
        half_i.append(mi)

    ba = half_v[0].astype(jnp.bfloat16).astype(jnp.float32)
    vb, ib = half_v[1], half_i[1]
    keep_a = ba < vb
    keep_a_idx = keep_a | ((ba == vb) & (half_i[0] < ib))
    mi = jnp.where(keep_a_idx, half_i[0], ib)

    for c in range(N_CHUNKS):
        iota = lax.broadcasted_iota(jnp.int32, (R, C), 1) + c * C
        oh = jnp.where(iota == mi, 1.0, 0.0).astype(jnp.float32)
        enc_ref[:, c * C:(c + 1) * C] = oh
        counts_ref[0:1, c * C:(c + 1) * C] += jnp.sum(oh, axis=0,
                                                      keepdims=True)

    idx_ref[...] = mi
    z_q = jnp.dot(enc_ref[...], emb_ref[...],
                  preferred_element_type=jnp.float32)
    z_q_st = zb + (z_q - zb)
    zq_ref[...] = z_q_st
    diff = z_q_st - zb
    acc_ref[...] += jnp.sum(diff * diff).reshape(1, 1)

    @pl.when(i == N_BLOCKS - 1)
    def _fin():
        lsum = acc_ref[0, 0]
        lmean = lsum / jnp.float32(N_VEC * DIM)
        loss_ref[...] = (lmean + BETA_ * lmean).reshape(1, 1)
        e_mean = counts_ref[...] / jnp.float32(N_VEC)
        plog = e_mean * jnp.log(e_mean + 1e-10)
        perp_ref[...] = jnp.exp(-jnp.sum(plog)).reshape(1, 1)


@functools.partial(jax.jit, static_argnames=())
def kernel(z, emb_w):
    zp = jnp.transpose(z, (0, 2, 3, 1))
    z_flat = zp.reshape(N_VEC, DIM)
    embT = emb_w.T
    embT2b = (2.0 * embT).astype(jnp.bfloat16)

    enc, zq, idx, loss, perp = pl.pallas_call(
        _vq_kernel,
        grid=(N_BLOCKS,),
        in_specs=[
            pl.BlockSpec((R, DIM), lambda i: (i, 0)),
            pl.BlockSpec((N_CODES, DIM), lambda i: (0, 0)),
            pl.BlockSpec((DIM, N_CODES), lambda i: (0, 0)),
            pl.BlockSpec((DIM, N_CODES), lambda i: (0, 0)),
        ],
        out_specs=[
            pl.BlockSpec((R, N_CODES), lambda i: (i, 0)),
            pl.BlockSpec((R, DIM), lambda i: (i, 0)),
            pl.BlockSpec((R, 1), lambda i: (i, 0)),
            pl.BlockSpec((1, 1), lambda i: (0, 0)),
            pl.BlockSpec((1, 1), lambda i: (0, 0)),
        ],
        out_shape=[
            jax.ShapeDtypeStruct((N_VEC, N_CODES), jnp.float32),
            jax.ShapeDtypeStruct((N_VEC, DIM), jnp.float32),
            jax.ShapeDtypeStruct((N_VEC, 1), jnp.int32),
            jax.ShapeDtypeStruct((1, 1), jnp.float32),
            jax.ShapeDtypeStruct((1, 1), jnp.float32),
        ],
        scratch_shapes=[
            pltpu.VMEM((1, N_CODES), jnp.float32),
            pltpu.VMEM((1, 1), jnp.float32),
        ],
    )(z_flat, emb_w, embT, embT2b)

    z_q_out = jnp.transpose(zq.reshape(zp.shape), (0, 3, 1, 2))
    return loss[0, 0], z_q_out, perp[0, 0], enc, idx


# speedup vs baseline: 1.1652x; 1.1652x over previous
import functools

import jax
import jax.numpy as jnp
from jax import lax
from jax.experimental import pallas as pl
from jax.experimental.pallas import tpu as pltpu

N_CODES = 8192
DIM = 32
N_VEC = 8192
R = 256
C = 512
N_BLOCKS = N_VEC // R
N_CHUNKS = N_CODES // C
HALF = N_CODES // 2
CHUNKS_PER_HALF = HALF // C
BETA_ = 0.25
MM_PREC = lax.Precision.HIGHEST


def _tree_sum_lanes(v):
    k = v.shape[-1]
    while k > 1:
        k //= 2
        v = v[..., :k] + v[..., k:2 * k]
    return v


def _tree_sum_rows(v):
    k = v.shape[0]
    while k > 1:
        k //= 2
        v = v[:k, :] + v[k:2 * k, :]
    return v


def _vq_kernel(z_ref, emb_ref, embT_ref, embT2b_ref,
               enc_ref, zq_ref, idx_ref, loss_ref, perp_ref,
               counts_ref, acc_ref):
    i = pl.program_id(0)
    zb = z_ref[...]
    zb16 = zb.astype(jnp.bfloat16)
    zsq = _tree_sum_lanes(zb * zb)
    embT = embT_ref[...]
    esq = _tree_sum_rows(embT * embT)

    @pl.when(i == 0)
    def _init():
        counts_ref[...] = jnp.zeros((1, N_CODES), jnp.float32)
        acc_ref[...] = jnp.zeros((1, 1), jnp.float32)

    half_v = []
    half_i = []
    for h in range(2):
        m = jnp.full((R, 1), jnp.inf, jnp.float32)
        mi = jnp.zeros((R, 1), jnp.int32)
        for cc in range(CHUNKS_PER_HALF):
            c = h * CHUNKS_PER_HALF + cc
            et2 = embT2b_ref[:, c * C:(c + 1) * C]
            mm2 = jnp.dot(zb16, et2,
                          preferred_element_type=jnp.float32)
            d = (zsq + esq[:, c * C:(c + 1) * C]) - mm2
            mc = jnp.min(d, axis=1, keepdims=True)
            iota = lax.broadcasted_iota(jnp.int32, (R, C), 1) + c * C
            ic = jnp.min(jnp.where(d == mc, iota, jnp.int32(2**30)),
                         axis=1, keepdims=True)
            upd = mc < m
            m = jnp.where(upd, mc, m)
            mi = jnp.where(upd, ic, mi)
        half_v.append(m)
        half_i.append(mi)

    ba = half_v[0].astype(jnp.bfloat16).astype(jnp.float32)
    vb, ib = half_v[1], half_i[1]
    keep_a = ba < vb
    keep_a_idx = keep_a | ((ba == vb) & (half_i[0] < ib))
    mi = jnp.where(keep_a_idx, half_i[0], ib)

    for c in range(N_CHUNKS):
        iota = lax.broadcasted_iota(jnp.int32, (R, C), 1) + c * C
        oh = jnp.where(iota == mi, 1.0, 0.0).astype(jnp.float32)
        enc_ref[:, c * C:(c + 1) * C] = oh
        counts_ref[0:1, c * C:(c + 1) * C] += jnp.sum(oh, axis=0,
                                                      keepdims=True)

    idx_ref[...] = mi
    z_q = jnp.dot(enc_ref[...], emb_ref[...],
                  preferred_element_type=jnp.float32)
    z_q_st = zb + (z_q - zb)
    zq_ref[...] = z_q_st
    diff = z_q_st - zb
    acc_ref[...] += jnp.sum(diff * diff).reshape(1, 1)

    @pl.when(i == N_BLOCKS - 1)
    def _fin():
        lsum = acc_ref[0, 0]
        lmean = lsum / jnp.float32(N_VEC * DIM)
        loss_ref[...] = (lmean + BETA_ * lmean).reshape(1, 1)
        e_mean = counts_ref[...] / jnp.float32(N_VEC)
        plog = e_mean * jnp.log(e_mean + 1e-10)
        perp_ref[...] = jnp.exp(-jnp.sum(plog)).reshape(1, 1)


@functools.partial(jax.jit, static_argnames=())
def kernel(z, emb_w):
    zp = jnp.transpose(z, (0, 2, 3, 1))
    z_flat = zp.reshape(N_VEC, DIM)
    embT = emb_w.T
    embT2b = (2.0 * embT).astype(jnp.bfloat16)

    enc, zq, idx, loss, perp = pl.pallas_call(
        _vq_kernel,
        grid=(N_BLOCKS,),
        in_specs=[
            pl.BlockSpec((R, DIM), lambda i: (i, 0)),
            pl.BlockSpec((N_CODES, DIM), lambda i: (0, 0)),
            pl.BlockSpec((DIM, N_CODES), lambda i: (0, 0)),
            pl.BlockSpec((DIM, N_CODES), lambda i: (0, 0)),
        ],
        out_specs=[
            pl.BlockSpec((R, N_CODES), lambda i: (i, 0)),
            pl.BlockSpec((R, DIM), lambda i: (i, 0)),
            pl.BlockSpec((R, 1), lambda i: (i, 0)),
            pl.BlockSpec((1, 1), lambda i: (0, 0)),
            pl.BlockSpec((1, 1), lambda i: (0, 0)),
        ],
        out_shape=[
            jax.ShapeDtypeStruct((N_VEC, N_CODES), jnp.float32),
            jax.ShapeDtypeStruct((N_VEC, DIM), jnp.float32),
            jax.ShapeDtypeStruct((N_VEC, 1), jnp.int32),
            jax.ShapeDtypeStruct((1, 1), jnp.float32),
            jax.ShapeDtypeStruct((1, 1), jnp.float32),
        ],
        scratch_shapes=[
            pltpu.VMEM((1, N_CODES), jnp.float32),
            pltpu.VMEM((1, 1), jnp.float32),
        ],
    )(z_flat, emb_w, embT, embT2b)

    z_q_out = jnp.transpose(zq.reshape(zp.shape), (0, 3, 1, 2))
    return loss[0, 0], z_q_out, perp[0, 0], enc, idx
